# baseline (speedup 1.0000x reference)
import functools

import jax
import jax.numpy as jnp
from jax import lax
from jax.experimental import pallas as pl
from jax.experimental.pallas import tpu as pltpu
from jax.experimental.pallas import tpu_sc as plsc

BLOCK_R = 4096
NUM_EXPERTS = 64
MODEL_DIM = 768
N_TOKENS = 32768
CHUNKS = (16384, 16384)

NC, NS, L = 2, 16, 16
N_WORKERS = NC * NS


def _matmul_body(x_ref, wt_ref, logits_ref, logits_t_ref):
    logits = jnp.dot(x_ref[...], wt_ref[...],
                     preferred_element_type=jnp.float32)
    logits_ref[...] = logits
    logits_t_ref[...] = logits.T


def _matmul_body_alias(x_ref, wt_ref, rm_prev_ref, logits_ref, logits_t_ref):
    del rm_prev_ref
    _matmul_body(x_ref, wt_ref, logits_ref, logits_t_ref)


def _tc_chunk(x, wt, row0, rows, rm_prev=None):
    steps = rows // BLOCK_R
    off = row0 // BLOCK_R
    in_specs = [
        pl.BlockSpec((BLOCK_R, MODEL_DIM), lambda i, o=off: (i + o, 0)),
        pl.BlockSpec((MODEL_DIM, NUM_EXPERTS), lambda i: (0, 0)),
    ]
    args = [x, wt]
    io_alias = {}
    body = _matmul_body
    if rm_prev is not None:
        in_specs.append(
            pl.BlockSpec((BLOCK_R, NUM_EXPERTS), lambda i, o=off: (i + o, 0)))
        args.append(rm_prev)
        io_alias = {2: 0}
        body = _matmul_body_alias
    return pl.pallas_call(
        body,
        grid=(steps,),
        in_specs=in_specs,
        out_specs=[
            pl.BlockSpec((BLOCK_R, NUM_EXPERTS), lambda i, o=off: (i + o, 0)),
            pl.BlockSpec((NUM_EXPERTS, BLOCK_R), lambda i: (0, i)),
        ],
        out_shape=[
            jax.ShapeDtypeStruct((N_TOKENS, NUM_EXPERTS), jnp.float32),
            jax.ShapeDtypeStruct((NUM_EXPERTS, rows), jnp.float32),
        ],
        input_output_aliases=io_alias,
    )(*args)


def _make_sc_route(chunk):
    rows_w = chunk // N_WORKERS

    @functools.partial(
        pl.kernel,
        out_type=[
            jax.ShapeDtypeStruct((2, chunk), jnp.float32),
            jax.ShapeDtypeStruct((2, chunk), jnp.int32),
        ],
        mesh=plsc.VectorSubcoreMesh(
            core_axis_name="c", subcore_axis_name="s", num_cores=NC,
            num_subcores=NS),
        scratch_types=[
            pltpu.VMEM((NUM_EXPERTS, rows_w), jnp.float32),
            pltpu.VMEM((rows_w,), jnp.float32),
            pltpu.VMEM((rows_w,), jnp.float32),
            pltpu.VMEM((rows_w,), jnp.int32),
            pltpu.VMEM((rows_w,), jnp.int32),
        ],
    )
    def _sc_route(lt_hbm, w_hbm, i_hbm, lt_v, w1_v, w2_v, i1_v, i2_v):
        wid = lax.axis_index("s") * NC + lax.axis_index("c")
        base = wid * rows_w
        pltpu.sync_copy(lt_hbm.at[:, pl.ds(base, rows_w)], lt_v)

        zeros = jnp.zeros((L,), jnp.int32)

        def group(g, carry):
            off = g * L
            m1 = lt_v[0, pl.ds(off, L)]
            i1 = zeros
            m2 = jnp.full((L,), -jnp.inf, jnp.float32)
            i2 = zeros
            for e in range(1, NUM_EXPERTS):
                e_v = jnp.full((L,), e, jnp.int32)
                v = lt_v[e, pl.ds(off, L)]
                gt1 = v > m1
                gt2 = v > m2
                m2 = jnp.where(gt1, m1, jnp.where(gt2, v, m2))
                i2 = jnp.where(gt1, i1, jnp.where(gt2, e_v, i2))
                m1 = jnp.where(gt1, v, m1)
                i1 = jnp.where(gt1, e_v, i1)
            s = jnp.exp(m2 - m1)
            d = 1.0 + s
            w1_v[pl.ds(off, L)] = 1.0 / d
            w2_v[pl.ds(off, L)] = s / d
            i1_v[pl.ds(off, L)] = i1
            i2_v[pl.ds(off, L)] = i2
            return carry

        lax.fori_loop(0, rows_w // L, group, 0)
        pltpu.sync_copy(w1_v, w_hbm.at[0, pl.ds(base, rows_w)])
        pltpu.sync_copy(w2_v, w_hbm.at[1, pl.ds(base, rows_w)])
        pltpu.sync_copy(i1_v, i_hbm.at[0, pl.ds(base, rows_w)])
        pltpu.sync_copy(i2_v, i_hbm.at[1, pl.ds(base, rows_w)])

    return _sc_route


_SC_ROUTES = {c: _make_sc_route(c) for c in sorted(set(CHUNKS))}


def kernel(x, W_gate, W_noise):
    del W_noise
    wt = W_gate.T
    rm = None
    row0 = 0
    w_chunks, i_chunks = [], []
    for chunk in CHUNKS:
        rm, lt = _tc_chunk(x, wt, row0, chunk, rm_prev=rm)
        w_planar, i_planar = _SC_ROUTES[chunk](lt)
        w_chunks.append(jnp.stack([w_planar[0], w_planar[1]], axis=-1))
        i_chunks.append(jnp.stack([i_planar[0], i_planar[1]], axis=-1))
        row0 += chunk
    weights = jnp.concatenate(w_chunks, axis=0)
    indices = jnp.concatenate(i_chunks, axis=0)
    return weights, indices, rm


# speedup vs baseline: 1.4679x; 1.4679x over previous
import jax
import jax.numpy as jnp
from jax import lax
from jax.experimental import pallas as pl

BLOCK_R = 4096
NUM_EXPERTS = 64
MODEL_DIM = 768
N_TOKENS = 32768


def _gating_body(x_ref, wt_ref, logits_ref, packed_ref):
    logits = jnp.dot(x_ref[...], wt_ref[...],
                     preferred_element_type=jnp.float32)
    logits_ref[...] = logits

    lt = logits.T
    iota_e = lax.broadcasted_iota(jnp.int32, lt.shape, 0)
    m1 = jnp.max(lt, axis=0, keepdims=True)
    i1 = jnp.min(jnp.where(lt == m1, iota_e, NUM_EXPERTS), axis=0,
                 keepdims=True)
    masked = jnp.where(iota_e == i1, -jnp.inf, lt)
    m2 = jnp.max(masked, axis=0, keepdims=True)
    i2 = jnp.min(jnp.where(masked == m2, iota_e, NUM_EXPERTS), axis=0,
                 keepdims=True)

    s = jnp.exp(m2 - m1)
    d = 1.0 + s
    w1 = 1.0 / d
    w2 = s / d
    i1f = lax.bitcast_convert_type(i1, jnp.float32)
    i2f = lax.bitcast_convert_type(i2, jnp.float32)
    packed_ref[...] = jnp.concatenate(
        [w1, w2, i1f, i2f, w1, w1, w1, w1], axis=0)


def kernel(x, W_gate, W_noise):
    del W_noise
    n = x.shape[0]
    wt = W_gate.T

    logits, packed = pl.pallas_call(
        _gating_body,
        grid=(n // BLOCK_R,),
        in_specs=[
            pl.BlockSpec((BLOCK_R, MODEL_DIM), lambda i: (i, 0)),
            pl.BlockSpec((MODEL_DIM, NUM_EXPERTS), lambda i: (0, 0)),
        ],
        out_specs=[
            pl.BlockSpec((BLOCK_R, NUM_EXPERTS), lambda i: (i, 0)),
            pl.BlockSpec((8, BLOCK_R), lambda i: (0, i)),
        ],
        out_shape=[
            jax.ShapeDtypeStruct((n, NUM_EXPERTS), jnp.float32),
            jax.ShapeDtypeStruct((8, n), jnp.float32),
        ],
    )(x, wt)

    weights = jnp.stack([packed[0], packed[1]], axis=-1)
    indices = jnp.stack(
        [lax.bitcast_convert_type(packed[2], jnp.int32),
         lax.bitcast_convert_type(packed[3], jnp.int32)], axis=-1)
    return weights, indices, logits
